# baseline (speedup 1.0000x reference)
import jax
import jax.numpy as jnp
from jax.experimental import pallas as pl
from jax.experimental.pallas import tpu as pltpu

N, E = 1024, 2048
NFEAT_V, NFEAT_E, NHID, NCLASS = 128, 16, 64, 16
BN = 256
BE = 256


def _bf(x):
    return x.astype(jnp.bfloat16).astype(jnp.float32)


def _fused_kernel(T_ref, Tc_ref, adj_v_ref, adj_e_ref, X_ref, Z_ref,
                  W1_ref, p1_ref, b1_ref, W2_ref, p2_ref, b2_ref,
                  W3_ref, p3_ref, b3_ref, out_ref,
                  Xh, Zh, Tbf, M, HWbf, ZWbf, d1, d2, d3, cmr_n, cmr_e):
    s = pl.program_id(0)
    buf = jax.lax.rem(s, 2)
    pbuf = jax.lax.rem(s + 1, 2)

    @pl.when(s == 0)
    def _init1():
        d1v = jnp.sum(_bf(Z_ref[...]) * _bf(p1_ref[...]), axis=1)
        d1[...] = d1v.reshape(1, E)
        Tbf[...] = T_ref[...].astype(jnp.bfloat16)
        HWbf[...] = jnp.dot(X_ref[...], W1_ref[...],
                            preferred_element_type=jnp.float32
                            ).astype(jnp.bfloat16)
        cmr_n[...] = (jax.lax.broadcasted_iota(jnp.int32, (BN, N), 1)
                      - jax.lax.broadcasted_iota(jnp.int32, (BN, N), 0))
        cmr_e[...] = (jax.lax.broadcasted_iota(jnp.int32, (BE, E), 1)
                      - jax.lax.broadcasted_iota(jnp.int32, (BE, E), 0))

    @pl.when(s < 4)
    def _c_gc1():
        i = s
        lhs = (T_ref[pl.ds(i * BN, BN), :] * d1[...]).astype(jnp.bfloat16)
        M[buf, :, :N] = jax.lax.dot_general(
            lhs, Tbf[...], (((1,), (1,)), ((), ())),
            preferred_element_type=jnp.float32)

    @pl.when((s >= 1) & (s < 5))
    def _d_gc1():
        i = s - 1
        A = jnp.where(cmr_n[...] == i * BN, 1.0, M[pbuf, :, :N]) \
            * adj_v_ref[...]
        out = jnp.dot(A.astype(jnp.bfloat16), HWbf[...],
                      preferred_element_type=jnp.float32) + b1_ref[...]
        Xh[pl.ds(i * BN, BN), :] = jnp.maximum(out, 0.0)

    @pl.when(s == 4)
    def _init2():
        d2[...] = jnp.sum(_bf(Xh[...]) * _bf(p2_ref[...]), axis=1,
                          keepdims=True)
        ZWbf[...] = jnp.dot(jnp.maximum(Z_ref[...], 0.0), W2_ref[...],
                            preferred_element_type=jnp.float32
                            ).astype(jnp.bfloat16)

    @pl.when((s >= 5) & (s < 13))
    def _c_gc2():
        M[buf] = jax.lax.dot_general(
            (Tc_ref[...] * d2[...]).astype(jnp.bfloat16), Tbf[...],
            (((0,), (0,)), ((), ())), preferred_element_type=jnp.float32)

    @pl.when((s >= 6) & (s < 14))
    def _d_gc2():
        j = s - 6
        A = jnp.where(cmr_e[...] == j * BE, 1.0, M[pbuf]) * adj_e_ref[...]
        out = jnp.dot(A.astype(jnp.bfloat16), ZWbf[...],
                      preferred_element_type=jnp.float32) + b2_ref[...]
        Zh[pl.ds(j * BE, BE), :] = jnp.maximum(out, 0.0)

    @pl.when(s == 13)
    def _init3():
        d3v = jnp.sum(_bf(Zh[...]) * _bf(p3_ref[...]), axis=1)
        d3[...] = d3v.reshape(1, E)
        HWbf[:, :NCLASS] = jnp.dot(Xh[...], W3_ref[...],
                                   preferred_element_type=jnp.float32
                                   ).astype(jnp.bfloat16)

    @pl.when((s >= 14) & (s < 18))
    def _c_gc3():
        i = s - 14
        lhs = (T_ref[pl.ds(i * BN, BN), :] * d3[...]).astype(jnp.bfloat16)
        M[buf, :, :N] = jax.lax.dot_general(
            lhs, Tbf[...], (((1,), (1,)), ((), ())),
            preferred_element_type=jnp.float32)

    @pl.when(s >= 15)
    def _d_gc3():
        i = s - 15
        A = jnp.where(cmr_n[...] == i * BN, 1.0, M[pbuf, :, :N]) \
            * adj_v_ref[...]
        out = jnp.dot(A.astype(jnp.bfloat16), HWbf[:, :NCLASS],
                      preferred_element_type=jnp.float32) + b3_ref[...]
        shifted = out - jnp.max(out, axis=1, keepdims=True)
        out_ref[...] = shifted - jnp.log(jnp.sum(jnp.exp(shifted), axis=1,
                                                 keepdims=True))


def kernel(X, Z, adj_e, adj_v, T, W1, p1, b1, W2, p2, b2, W3, p3, b3):
    b1r, b2r, b3r = b1.reshape(1, -1), b2.reshape(1, -1), b3.reshape(1, -1)
    const = lambda a, b: (lambda s: (a, b))

    def adj_v_map(s):
        return (jnp.where(s < 5, jnp.clip(s - 1, 0, 3),
                          jnp.clip(s - 15, 0, 3)), 0)

    return pl.pallas_call(
        _fused_kernel,
        grid=(19,),
        in_specs=[
            pl.BlockSpec((N, E), const(0, 0)),
            pl.BlockSpec((N, BE), lambda s: (0, jnp.clip(s - 5, 0, 7))),
            pl.BlockSpec((BN, N), adj_v_map),
            pl.BlockSpec((BE, E), lambda s: (jnp.clip(s - 6, 0, 7), 0)),
            pl.BlockSpec((N, NFEAT_V), const(0, 0)),
            pl.BlockSpec((E, NFEAT_E), const(0, 0)),
            pl.BlockSpec((NFEAT_V, NHID), const(0, 0)),
            pl.BlockSpec((1, NFEAT_E), const(0, 0)),
            pl.BlockSpec((1, NHID), const(0, 0)),
            pl.BlockSpec((NFEAT_E, NFEAT_E), const(0, 0)),
            pl.BlockSpec((1, NHID), const(0, 0)),
            pl.BlockSpec((1, NFEAT_E), const(0, 0)),
            pl.BlockSpec((NHID, NCLASS), const(0, 0)),
            pl.BlockSpec((1, NFEAT_E), const(0, 0)),
            pl.BlockSpec((1, NCLASS), const(0, 0)),
        ],
        out_specs=pl.BlockSpec((BN, NCLASS),
                               lambda s: (jnp.clip(s - 15, 0, 3), 0)),
        out_shape=jax.ShapeDtypeStruct((N, NCLASS), jnp.float32),
        scratch_shapes=[
            pltpu.VMEM((N, NHID), jnp.float32),
            pltpu.VMEM((E, NFEAT_E), jnp.float32),
            pltpu.VMEM((N, E), jnp.bfloat16),
            pltpu.VMEM((2, BE, E), jnp.float32),
            pltpu.VMEM((N, NHID), jnp.bfloat16),
            pltpu.VMEM((E, NFEAT_E), jnp.bfloat16),
            pltpu.VMEM((1, E), jnp.float32),
            pltpu.VMEM((N, 1), jnp.float32),
            pltpu.VMEM((1, E), jnp.float32),
            pltpu.VMEM((BN, N), jnp.int32),
            pltpu.VMEM((BE, E), jnp.int32),
        ],
    )(T, T, adj_v, adj_e, X, Z, W1, p1, b1r, W2, p2, b2r, W3, p3, b3r)
